# baseline (speedup 1.0000x reference)
import functools

import jax
import jax.numpy as jnp
from jax import lax
from jax.experimental import pallas as pl
from jax.experimental.pallas import tpu as pltpu
from jax.experimental.pallas import tpu_sc as plsc

_INFO = plsc.get_sparse_core_info()
_NC = _INFO.num_cores
_NS = _INFO.num_subcores
_NW = _NC * _NS

_CHUNK = 128


def _gather_transposed(table, idx_t):
    nh, nb = idx_t.shape
    V, D = table.shape
    b_per_w = nb // _NW
    n_half = b_per_w // _CHUNK
    n_chunks = nh * n_half
    pstride = D + 1
    mesh = plsc.VectorSubcoreMesh(core_axis_name="c", subcore_axis_name="s")

    @functools.partial(
        pl.kernel,
        mesh=mesh,
        out_type=jax.ShapeDtypeStruct((nh, D // 8, nb // 128, 8, 128), jnp.float32),
        scratch_types=[
            pltpu.VMEM((nh, b_per_w), jnp.int32),
            pltpu.VMEM((2, _CHUNK, D), jnp.float32),
            pltpu.VMEM((_CHUNK * pstride,), jnp.float32),
            pltpu.VMEM((2, D // 8, 1, 8, 128), jnp.float32),
            [pltpu.SemaphoreType.DMA] * 2,
            [pltpu.SemaphoreType.DMA] * 2,
        ],
        compiler_params=pltpu.CompilerParams(
            use_tc_tiling_on_sc=False, needs_layout_passes=False
        ),
    )
    def k(table_hbm, idx_hbm, out_hbm, idx_v, rows_v, pad_v, col_v, gsem, osem):
        wid = lax.axis_index("s") * _NC + lax.axis_index("c")
        b0 = wid * b_per_w

        def start_gather(g, p):
            h = g // n_half
            half = g % n_half
            pltpu.async_copy(
                table_hbm.at[idx_v.at[h, pl.ds(half * _CHUNK, _CHUNK)]],
                rows_v.at[p],
                gsem[p],
            )

        def wait_gather(p):
            pltpu.make_async_copy(
                table_hbm.at[pl.ds(0, _CHUNK)], rows_v.at[p], gsem[p]
            ).wait()

        def wait_wb(p):
            pltpu.make_async_copy(
                out_hbm.at[0, :, pl.ds(0, 1), :, :],
                col_v.at[p],
                osem[p],
            ).wait()

        pltpu.sync_copy(idx_hbm.at[:, pl.ds(b0, b_per_w)], idx_v)
        for p in range(2):
            start_gather(p, p)

        def body(i, carry):
          for p in range(2):
            g = i * 2 + p
            h = g // n_half
            half = g % n_half
            wait_gather(p)

            iota16 = lax.iota(jnp.int32, 16)
            for b in range(_CHUNK):
                for dg in range(D // 16):
                    off = b * pstride + dg * 16
                    off8 = (off // 8) * 8
                    plsc.store_scatter(
                        pad_v.at[pl.ds(off8, 24)],
                        [iota16 + (off - off8)],
                        rows_v[p, b, pl.ds(dg * 16, 16)],
                    )

            @pl.when(g + 2 < n_chunks)
            def _():
                start_gather(g + 2, p)

            @pl.when(g >= 2)
            def _():
                wait_wb(p)

            iota65 = iota16 * pstride
            for lb in range(8):
                for d in range(D):
                    off = lb * 16 * pstride + d
                    off8 = (off // 8) * 8
                    vals = plsc.load_gather(
                        pad_v.at[pl.ds(off8, 15 * pstride + 8)],
                        [iota65 + (off - off8)],
                    )
                    col_v[p, d // 8, 0, d % 8, pl.ds(lb * 16, 16)] = vals

            c0 = wid * (b_per_w // 128) + half
            pltpu.async_copy(
                col_v.at[p],
                out_hbm.at[h, :, pl.ds(c0, 1), :, :],
                osem[p],
            )

          return carry

        lax.fori_loop(0, n_chunks // 2, body, 0)

        for p in range(2):
            wait_wb(p)

    return k(table, idx_t)


def kernel(indices, table):
    nb, nh = indices.shape
    V, D = table.shape
    out5 = _gather_transposed(table, indices.T)
    return out5.transpose((2, 4, 0, 1, 3)).reshape(nb, nh, D)


# speedup vs baseline: 1.5424x; 1.5424x over previous
import functools

import jax
import jax.numpy as jnp
from jax import lax
from jax.experimental import pallas as pl
from jax.experimental.pallas import tpu as pltpu
from jax.experimental.pallas import tpu_sc as plsc

_INFO = plsc.get_sparse_core_info()
_NC = _INFO.num_cores
_NS = _INFO.num_subcores
_NW = _NC * _NS

_CHUNK = 256


def _gather_transposed(table, idx_t):
    nh, nb = idx_t.shape
    V, D = table.shape
    b_per_w = nb // _NW
    n_half = b_per_w // _CHUNK
    n_chunks = nh * n_half
    c_per_chunk = _CHUNK // 128
    pstride = D + 1
    mesh = plsc.VectorSubcoreMesh(core_axis_name="c", subcore_axis_name="s")

    @functools.partial(
        pl.kernel,
        mesh=mesh,
        out_type=jax.ShapeDtypeStruct((nh, D // 8, nb // 128, 8, 128), jnp.float32),
        scratch_types=[
            pltpu.VMEM((nh, b_per_w), jnp.int32),
            pltpu.VMEM((2, _CHUNK, D), jnp.float32),
            pltpu.VMEM((_CHUNK * pstride,), jnp.float32),
            pltpu.VMEM((2, D // 8, c_per_chunk, 8, 128), jnp.float32),
            [pltpu.SemaphoreType.DMA] * 2,
            [pltpu.SemaphoreType.DMA] * 2,
        ],
        compiler_params=pltpu.CompilerParams(
            use_tc_tiling_on_sc=False, needs_layout_passes=False
        ),
    )
    def k(table_hbm, idx_hbm, out_hbm, idx_v, rows_v, pad_v, col_v, gsem, osem):
        wid = lax.axis_index("s") * _NC + lax.axis_index("c")
        b0 = wid * b_per_w

        def start_gather(g, p):
            h = g // n_half
            half = g % n_half
            pltpu.async_copy(
                table_hbm.at[idx_v.at[h, pl.ds(half * _CHUNK, _CHUNK)]],
                rows_v.at[p],
                gsem[p],
            )

        def wait_gather(p):
            pltpu.make_async_copy(
                table_hbm.at[pl.ds(0, _CHUNK)], rows_v.at[p], gsem[p]
            ).wait()

        def wait_wb(p):
            pltpu.make_async_copy(
                out_hbm.at[0, :, pl.ds(0, c_per_chunk), :, :],
                col_v.at[p],
                osem[p],
            ).wait()

        pltpu.sync_copy(idx_hbm.at[:, pl.ds(b0, b_per_w)], idx_v)
        for p in range(2):
            start_gather(p, p)

        def body(i, carry):
          for p in range(2):
            g = i * 2 + p
            h = g // n_half
            half = g % n_half
            wait_gather(p)

            iota16 = lax.iota(jnp.int32, 16)

            @plsc.parallel_loop(0, _CHUNK // 8, 1, unroll=2)
            def _(bb):
                for j in range(8):
                    b = bb * 8 + j
                    for dg in range(D // 16):
                        pad_v[pl.ds(b * pstride + dg * 16, 16)] = rows_v[
                            p, b, pl.ds(dg * 16, 16)
                        ]

            @pl.when(g + 2 < n_chunks)
            def _():
                start_gather(g + 2, p)

            @pl.when(g >= 2)
            def _():
                wait_wb(p)

            for c2 in range(c_per_chunk):

                @plsc.parallel_loop(0, 8, 1, unroll=2)
                def _(lb, c2=c2):
                    base = (iota16 + (c2 * 128 + lb * 16)) * pstride
                    for d in range(D):
                        vals = plsc.load_gather(pad_v, [base + d])
                        col_v[p, d // 8, c2, d % 8, pl.ds(lb * 16, 16)] = vals

            c0 = wid * (b_per_w // 128) + half * c_per_chunk
            pltpu.async_copy(
                col_v.at[p],
                out_hbm.at[h, :, pl.ds(c0, c_per_chunk), :, :],
                osem[p],
            )

          return carry

        lax.fori_loop(0, n_chunks // 2, body, 0)

        for p in range(2):
            wait_wb(p)

    return k(table, idx_t)


def kernel(indices, table):
    nb, nh = indices.shape
    V, D = table.shape
    out5 = _gather_transposed(table, indices.T)
    return out5.transpose((2, 4, 0, 1, 3)).reshape(nb, nh, D)


# speedup vs baseline: 1.8684x; 1.2113x over previous
import functools

import jax
import jax.numpy as jnp
from jax import lax
from jax.experimental import pallas as pl
from jax.experimental.pallas import tpu as pltpu
from jax.experimental.pallas import tpu_sc as plsc

_INFO = plsc.get_sparse_core_info()
_NC = _INFO.num_cores
_NS = _INFO.num_subcores
_NW = _NC * _NS

_CHUNK = 256


def _gather_transposed(table, idx_t):
    nh, nb = idx_t.shape
    V, D = table.shape
    b_per_w = nb // _NW
    n_half = b_per_w // _CHUNK
    n_chunks = nh * n_half
    c_per_chunk = _CHUNK // 128
    pstride = D + 1
    mesh = plsc.VectorSubcoreMesh(core_axis_name="c", subcore_axis_name="s")

    @functools.partial(
        pl.kernel,
        mesh=mesh,
        out_type=jax.ShapeDtypeStruct((nh, D // 8, nb // 128, 8, 128), jnp.float32),
        scratch_types=[
            pltpu.VMEM((nh, b_per_w), jnp.int32),
            pltpu.VMEM((2, _CHUNK, D), jnp.float32),
            pltpu.VMEM((_CHUNK * pstride,), jnp.float32),
            pltpu.VMEM((2, D // 8, c_per_chunk, 8, 128), jnp.float32),
            [pltpu.SemaphoreType.DMA] * 2,
            [pltpu.SemaphoreType.DMA] * 2,
        ],
        compiler_params=pltpu.CompilerParams(
            use_tc_tiling_on_sc=False, needs_layout_passes=False
        ),
    )
    def k(table_hbm, idx_hbm, out_hbm, idx_v, rows_v, pad_v, col_v, gsem, osem):
        wid = lax.axis_index("s") * _NC + lax.axis_index("c")
        b0 = wid * b_per_w

        def start_gather(g, p):
            h = g // n_half
            half = g % n_half
            pltpu.async_copy(
                table_hbm.at[idx_v.at[h, pl.ds(half * _CHUNK, _CHUNK)]],
                rows_v.at[p],
                gsem[p],
            )

        def wait_gather(p):
            pltpu.make_async_copy(
                table_hbm.at[pl.ds(0, _CHUNK)], rows_v.at[p], gsem[p]
            ).wait()

        def wait_wb(p):
            pltpu.make_async_copy(
                out_hbm.at[0, :, pl.ds(0, c_per_chunk), :, :],
                col_v.at[p],
                osem[p],
            ).wait()

        pltpu.sync_copy(idx_hbm.at[:, pl.ds(b0, b_per_w)], idx_v)
        for p in range(2):
            start_gather(p, p)

        def body(i, carry):
          for p in range(2):
            g = i * 2 + p
            h = g // n_half
            half = g % n_half
            wait_gather(p)

            iota16 = lax.iota(jnp.int32, 16)

            @plsc.parallel_loop(0, _CHUNK // 8, 1, unroll=4)
            def _(bb):
                for j in range(8):
                    b = bb * 8 + j
                    for dg in range(D // 16):
                        pad_v[pl.ds(b * pstride + dg * 16, 16)] = rows_v[
                            p, b, pl.ds(dg * 16, 16)
                        ]

            @pl.when(g + 2 < n_chunks)
            def _():
                start_gather(g + 2, p)

            @pl.when(g >= 2)
            def _():
                wait_wb(p)

            for c2 in range(c_per_chunk):

                @plsc.parallel_loop(0, 8, 1, unroll=4)
                def _(lb, c2=c2):
                    base = (iota16 + (c2 * 128 + lb * 16)) * pstride
                    for d in range(D):
                        vals = plsc.load_gather(pad_v, [base + d])
                        col_v[p, d // 8, c2, d % 8, pl.ds(lb * 16, 16)] = vals

            c0 = wid * (b_per_w // 128) + half * c_per_chunk
            pltpu.async_copy(
                col_v.at[p],
                out_hbm.at[h, :, pl.ds(c0, c_per_chunk), :, :],
                osem[p],
            )

          return carry

        lax.fori_loop(0, n_chunks // 2, body, 0)

        for p in range(2):
            wait_wb(p)

    return k(table, idx_t)


def kernel(indices, table):
    nb, nh = indices.shape
    V, D = table.shape
    out5 = _gather_transposed(table, indices.T)
    return out5.transpose((2, 4, 0, 1, 3)).reshape(nb, nh, D)
